# baseline (speedup 1.0000x reference)
import functools

import jax
import jax.numpy as jnp
from jax import lax
from jax.experimental import pallas as pl
from jax.experimental.pallas import tpu as pltpu
from jax.experimental.pallas import tpu_sc as plsc

NC = 2
NS = 16
L = 16
NW = NC * NS

B = 16384
D = 64
BPW = B // NW
NCHUNK = 4
CHUNK = BPW // NCHUNK
NGROUP = BPW // L


_K1_COLS = 8192
_HALF_BLOCKS = 62
HALF_TC = _K1_COLS * _HALF_BLOCKS
_LAST_BLOCK = 122


def _relayout_body(lo_ref, hi_ref, out_ref):
    stacked = jnp.concatenate([lo_ref[...], hi_ref[...]], axis=0)
    ii = lax.broadcasted_iota(jnp.int32, (128, 128), 0)
    jj = lax.broadcasted_iota(jnp.int32, (128, 128), 1)
    eye = (ii == jj).astype(jnp.float32)
    out_ref[...] = lax.dot_general(
        stacked, eye, (((0,), (0,)), ((), ())),
        preferred_element_type=jnp.float32,
        precision=lax.Precision.HIGHEST,
    )


def _relayout_tc(table_t):
    return pl.pallas_call(
        _relayout_body,
        grid=(_HALF_BLOCKS,),
        in_specs=[
            pl.BlockSpec((D, _K1_COLS), lambda i: (0, i)),
            pl.BlockSpec((D, _K1_COLS),
                         lambda i: (0, jnp.minimum(i + _HALF_BLOCKS,
                                                   _LAST_BLOCK))),
        ],
        out_specs=pl.BlockSpec((_K1_COLS, 128), lambda i: (i, 0)),
        out_shape=jax.ShapeDtypeStruct((HALF_TC, 128), jnp.float32),
        compiler_params=pltpu.CompilerParams(
            fuse_transposed_lhs_in_matmul=True),
    )(table_t, table_t)


TAIL_BASE = 999936


def _tail_body(t_ref, out_ref):
    x = t_ref[...].T
    out_ref[...] = jnp.concatenate([x, x], axis=1)


def _tail_tc(table_t):
    return pl.pallas_call(
        _tail_body,
        grid=(1,),
        in_specs=[pl.BlockSpec((D, 128), lambda i: (0, TAIL_BASE // 128))],
        out_specs=pl.BlockSpec((128, 128), lambda i: (0, 0)),
        out_shape=jax.ShapeDtypeStruct((128, 128), jnp.float32),
    )(table_t)


CT_HALF = 3906
HALF_SC = CT_HALF * 128
_PAIRS_PW = 124

_mesh = plsc.VectorSubcoreMesh(
    core_axis_name="c", subcore_axis_name="s", num_cores=NC, num_subcores=NS
)

_LANE = None


def _transpose_block(xs, lane):
    xs = list(xs)
    for d in (8, 4, 2, 1):
        mask = (lane & d) == 0
        perm = lane ^ d
        for k in range(L):
            if k & d:
                continue
            j = k | d
            a, b = xs[k], xs[j]
            ash = a.at[perm].get(mode="promise_in_bounds")
            bsh = b.at[perm].get(mode="promise_in_bounds")
            xs[k] = jnp.where(mask, a, bsh)
            xs[j] = jnp.where(mask, ash, b)
    return xs


@functools.partial(
    pl.kernel,
    out_type=jax.ShapeDtypeStruct((HALF_SC, 128), jnp.float32),
    mesh=_mesh,
    scratch_types=[
        pltpu.VMEM((8, 8, 128), jnp.float32),
        pltpu.VMEM((8, 8, 128), jnp.float32),
        pltpu.VMEM((8, 8, 128), jnp.float32),
        pltpu.VMEM((8, 8, 128), jnp.float32),
        pltpu.VMEM((128, 128), jnp.float32),
        pltpu.VMEM((128, 128), jnp.float32),
        pltpu.SemaphoreType.DMA,
        pltpu.SemaphoreType.DMA,
        pltpu.SemaphoreType.DMA,
        pltpu.SemaphoreType.DMA,
    ],
)
def _convert_sc(it3_hbm, out_hbm, plo0, phi0, plo1, phi1, ob0, ob1,
                si0, si1, so0, so1):
    wid = lax.axis_index("s") * NC + lax.axis_index("c")
    base = wid * _PAIRS_PW
    lane = lax.iota(jnp.int32, L)
    cmax = jnp.int32(CT_HALF - 1)

    banks = [(plo0, phi0, ob0, si0, so0), (plo1, phi1, ob1, si1, so1)]

    def fire(c, bank):
        plo, phi, _, si, _ = banks[bank]
        cc = pl.multiple_of(jnp.minimum(c, cmax) * 128, 128)
        ch = pl.multiple_of(
            (jnp.minimum(c, cmax) + CT_HALF) * 128, 128)
        pltpu.async_copy(it3_hbm.at[:, :, pl.ds(cc, 128)], plo, si)
        pltpu.async_copy(it3_hbm.at[:, :, pl.ds(ch, 128)], phi, si)

    def wait_in(bank):
        plo, phi, _, si, _ = banks[bank]
        pltpu.make_async_copy(
            it3_hbm.at[:, :, pl.ds(0, 128)], plo, si).wait()
        pltpu.make_async_copy(
            it3_hbm.at[:, :, pl.ds(0, 128)], phi, si).wait()

    def wait_out(bank):
        _, _, ob, _, so = banks[bank]
        pltpu.make_async_copy(ob, out_hbm.at[pl.ds(0, 128)], so).wait()

    def transpose_panel(pv, ob, half):
        def blk_body(blk, carry):
            fg = blk >> 3
            gi = blk & 7
            xs = [pv[2 * fg + (k >> 3), k & 7, pl.ds(gi * L, L)]
                  for k in range(L)]
            ys = _transpose_block(xs, lane)
            for j in range(L):
                ob[gi * L + j, pl.ds(half * 64 + fg * L, L)] = ys[j]
            return carry
        lax.fori_loop(0, 32, blk_body, 0)

    fire(base, 0)
    fire(base + 1, 1)

    def iter_body(q, carry):
        for bank in range(2):
            plo, phi, ob, _, so = banks[bank]
            c = base + 2 * q + bank
            wait_in(bank)

            @pl.when(q > 0)
            def _():
                wait_out(bank)

            transpose_panel(plo, ob, 0)
            transpose_panel(phi, ob, 1)
            cc = pl.multiple_of(jnp.minimum(c, cmax) * 128, 128)
            pltpu.async_copy(ob, out_hbm.at[pl.ds(cc, 128)], so)
            fire(c + 2, bank)
        return carry

    lax.fori_loop(0, _PAIRS_PW // 2, iter_body, 0)

    for bank in range(2):
        wait_in(bank)
        wait_out(bank)


@functools.partial(
    pl.kernel,
    out_type=jax.ShapeDtypeStruct((NW, BPW), jnp.float32),
    mesh=_mesh,
    scratch_types=[
        pltpu.VMEM((NCHUNK, CHUNK), jnp.int32),
        pltpu.VMEM((NCHUNK, CHUNK), jnp.int32),
        pltpu.VMEM((NCHUNK, CHUNK), jnp.int32),
        pltpu.VMEM((NCHUNK, CHUNK), jnp.int32),
        pltpu.VMEM((NCHUNK, CHUNK), jnp.int32),
        pltpu.VMEM((NGROUP, L), jnp.int32),
        pltpu.VMEM((NGROUP, L), jnp.int32),
        pltpu.VMEM((CHUNK, 128), jnp.float32),
        pltpu.VMEM((CHUNK, 128), jnp.float32),
        pltpu.VMEM((CHUNK, 128), jnp.float32),
        pltpu.VMEM((CHUNK, 128), jnp.float32),
        pltpu.VMEM((CHUNK, 128), jnp.float32),
        pltpu.VMEM((CHUNK, 128), jnp.float32),
        pltpu.VMEM((BPW,), jnp.float32),
        pltpu.VMEM((BPW,), jnp.float32),
        pltpu.VMEM((BPW,), jnp.float32),
        pltpu.SemaphoreType.DMA,
        pltpu.SemaphoreType.DMA,
        pltpu.SemaphoreType.DMA,
    ],
)
def _scores_kernel(uid4_hbm, iid4_hbm, urid_hbm, imid_hbm, itid_hbm,
                   uidr_hbm, iidr_hbm, uer_hbm, ier_hbm, itail_hbm,
                   ub_hbm, ib_hbm,
                   out_hbm, uid_v, iid_v, urid_v, imid_v, itid_v,
                   uidr_v, iidr_v, ue_a, ue_b, ie_a, ie_b, it_a, it_b,
                   ub_v, ib_v, out_v, sem_a, sem_b, sem):
    wid = lax.axis_index("s") * NC + lax.axis_index("c")

    pltpu.sync_copy(uid4_hbm.at[wid], uid_v)
    pltpu.sync_copy(iid4_hbm.at[wid], iid_v)
    pltpu.sync_copy(urid_hbm.at[wid], urid_v)
    pltpu.sync_copy(imid_hbm.at[wid], imid_v)
    pltpu.sync_copy(itid_hbm.at[wid], itid_v)
    pltpu.sync_copy(uidr_hbm.at[wid], uidr_v)
    pltpu.sync_copy(iidr_hbm.at[wid], iidr_v)

    bias_copies = []
    for c in range(NCHUNK):
        sl = pl.ds(c * CHUNK, CHUNK)
        bias_copies.append(
            pltpu.async_copy(ub_hbm.at[uid_v.at[c]], ub_v.at[sl], sem))
        bias_copies.append(
            pltpu.async_copy(ib_hbm.at[iid_v.at[c]], ib_v.at[sl], sem))

    bufs = [(ue_a, ie_a, it_a, sem_a), (ue_b, ie_b, it_b, sem_b)]

    def fire(c):
        ue, ie, it, s = bufs[c % 2]
        return (pltpu.async_copy(uer_hbm.at[urid_v.at[c]], ue, s),
                pltpu.async_copy(ier_hbm.at[imid_v.at[c]], ie, s),
                pltpu.async_copy(itail_hbm.at[itid_v.at[c]], it, s))

    lane = lax.iota(jnp.int32, L)
    half_tc = jnp.full((L,), HALF_TC, jnp.int32)
    half_sc = jnp.full((L,), HALF_SC, jnp.int32)
    tail_b = jnp.full((L,), TAIL_BASE, jnp.int32)
    c64 = jnp.full((L,), 64, jnp.int32)
    c0 = jnp.zeros((L,), jnp.int32)
    c1 = jnp.full((L,), 1, jnp.int32)

    def hsum(v):
        for dist in (8, 4, 2, 1):
            v = v + v.at[lane ^ dist].get(mode="promise_in_bounds")
        return v

    gpc = CHUNK // L

    cur = fire(0)
    for c in range(NCHUNK):
        nxt = fire(c + 1) if c + 1 < NCHUNK else None
        for h in cur:
            h.wait()
        if c == 0:
            for cp in bias_copies:
                cp.wait()
        ue_v, ie_v, it_v, _ = bufs[c % 2]

        def body(gi, carry, c=c, ue_v=ue_v, ie_v=ie_v, it_v=it_v):
            g = c * gpc + gi
            row0 = gi * L
            uid16 = uidr_v[g]
            iid16 = iidr_v[g]
            hvu = jnp.where(uid16 >= half_tc, c64, c0)
            hvi = jnp.where((iid16 >= half_sc) & (iid16 < tail_b), c64, c0)
            tvi = jnp.where(iid16 >= tail_b, c1, c0)
            res = jnp.zeros((L,), jnp.float32)
            for k in range(L):
                hu = hvu[k]
                hi = hvi[k]
                tb = tvi[k]
                r = row0 + k
                acc = None
                for cc in range(D // L):
                    u = ue_v[r, pl.ds(hu + cc * L, L)]
                    vm = ie_v[r, pl.ds(hi + cc * L, L)]
                    vt = it_v[r, pl.ds(cc * L, L)]
                    v = jnp.where(tb == 1, vt, vm)
                    term = u * v
                    acc = term if acc is None else acc + term
                res = jnp.where(lane == k, hsum(acc), res)
            sl = pl.ds(g * L, L)
            out_v[sl] = res + ub_v[sl] + ib_v[sl]
            return carry

        lax.fori_loop(0, gpc, body, 0)
        cur = nxt

    pltpu.sync_copy(out_v, out_hbm.at[wid])


def kernel(user_ids, item_ids, user_embed, item_embed, user_bias, item_bias):
    uids = user_ids.astype(jnp.int32)
    iids = item_ids.astype(jnp.int32)

    uer = _relayout_tc(user_embed.T)
    itail = _tail_tc(item_embed.T)
    it3 = item_embed.T.reshape(8, 8, item_embed.shape[0])
    ier = _convert_sc(it3)

    urid = uids % HALF_TC
    is_tail = iids >= TAIL_BASE
    imid = jnp.where(is_tail, 0, iids % HALF_SC)
    itid = jnp.where(is_tail, iids - TAIL_BASE, 0)

    out = _scores_kernel(
        uids.reshape(NW, NCHUNK, CHUNK),
        iids.reshape(NW, NCHUNK, CHUNK),
        urid.reshape(NW, NCHUNK, CHUNK),
        imid.reshape(NW, NCHUNK, CHUNK),
        itid.reshape(NW, NCHUNK, CHUNK),
        uids.reshape(NW, NGROUP, L),
        iids.reshape(NW, NGROUP, L),
        uer,
        ier,
        itail,
        user_bias.reshape(-1),
        item_bias.reshape(-1),
    )
    return out.reshape(B)


# speedup vs baseline: 2.4205x; 2.4205x over previous
import functools

import jax
import jax.numpy as jnp
from jax import lax
from jax.experimental import pallas as pl
from jax.experimental.pallas import tpu as pltpu
from jax.experimental.pallas import tpu_sc as plsc

NC = 2
NS = 16
L = 16
NW = NC * NS

B = 16384
D = 64
BPW = B // NW
NCHUNK = 4
CHUNK = BPW // NCHUNK
NGROUP = BPW // L


_K1_COLS = 8192
_HALF_BLOCKS = 62
HALF_TC = _K1_COLS * _HALF_BLOCKS
_LAST_BLOCK = 122


def _relayout_body(lo_ref, hi_ref, out_ref):
    stacked = jnp.concatenate([lo_ref[...], hi_ref[...]], axis=0)
    ii = lax.broadcasted_iota(jnp.int32, (128, 128), 0)
    jj = lax.broadcasted_iota(jnp.int32, (128, 128), 1)
    eye = (ii == jj).astype(jnp.float32)
    out_ref[...] = lax.dot_general(
        stacked, eye, (((0,), (0,)), ((), ())),
        preferred_element_type=jnp.float32,
        precision=lax.Precision.HIGHEST,
    )


def _relayout_tc(table_t):
    return pl.pallas_call(
        _relayout_body,
        grid=(_HALF_BLOCKS,),
        in_specs=[
            pl.BlockSpec((D, _K1_COLS), lambda i: (0, i)),
            pl.BlockSpec((D, _K1_COLS),
                         lambda i: (0, jnp.minimum(i + _HALF_BLOCKS,
                                                   _LAST_BLOCK))),
        ],
        out_specs=pl.BlockSpec((_K1_COLS, 128), lambda i: (i, 0)),
        out_shape=jax.ShapeDtypeStruct((HALF_TC, 128), jnp.float32),
        compiler_params=pltpu.CompilerParams(
            fuse_transposed_lhs_in_matmul=True),
    )(table_t, table_t)


TAIL_BASE = 999936


def _tail_body(t_ref, out_ref):
    x = t_ref[...].T
    out_ref[...] = jnp.concatenate([x, x], axis=1)


def _tail_tc(table_t):
    return pl.pallas_call(
        _tail_body,
        grid=(1,),
        in_specs=[pl.BlockSpec((D, 128), lambda i: (0, TAIL_BASE // 128))],
        out_specs=pl.BlockSpec((128, 128), lambda i: (0, 0)),
        out_shape=jax.ShapeDtypeStruct((128, 128), jnp.float32),
    )(table_t)


CT_HALF = 3906
HALF_SC = CT_HALF * 128
_PAIRS_PW = 124

_mesh = plsc.VectorSubcoreMesh(
    core_axis_name="c", subcore_axis_name="s", num_cores=NC, num_subcores=NS
)

_LANE = None


def _transpose_block(xs, lane):
    xs = list(xs)
    for d in (8, 4, 2, 1):
        mask = (lane & d) == 0
        perm = lane ^ d
        for k in range(L):
            if k & d:
                continue
            j = k | d
            a, b = xs[k], xs[j]
            ash = a.at[perm].get(mode="promise_in_bounds")
            bsh = b.at[perm].get(mode="promise_in_bounds")
            xs[k] = jnp.where(mask, a, bsh)
            xs[j] = jnp.where(mask, ash, b)
    return xs


@functools.partial(
    pl.kernel,
    out_type=jax.ShapeDtypeStruct((HALF_SC + 128, 128), jnp.float32),
    mesh=_mesh,
    scratch_types=[
        pltpu.VMEM((8, 8, 128), jnp.float32),
        pltpu.VMEM((8, 8, 128), jnp.float32),
        pltpu.VMEM((8, 8, 128), jnp.float32),
        pltpu.VMEM((8, 8, 128), jnp.float32),
        pltpu.VMEM((128, 128), jnp.float32),
        pltpu.VMEM((128, 128), jnp.float32),
        pltpu.SemaphoreType.DMA,
        pltpu.SemaphoreType.DMA,
        pltpu.SemaphoreType.DMA,
        pltpu.SemaphoreType.DMA,
    ],
)
def _convert_sc(it3_hbm, itail_hbm, out_hbm, plo0, phi0, plo1, phi1,
                ob0, ob1, si0, si1, so0, so1):
    wid = lax.axis_index("s") * NC + lax.axis_index("c")
    base = wid * _PAIRS_PW
    lane = lax.iota(jnp.int32, L)
    cmax = jnp.int32(CT_HALF - 1)

    pltpu.sync_copy(itail_hbm, ob0)
    pltpu.sync_copy(ob0, out_hbm.at[pl.ds(HALF_SC, 128)])

    banks = [(plo0, phi0, ob0, si0, so0), (plo1, phi1, ob1, si1, so1)]

    def fire(c, bank):
        plo, phi, _, si, _ = banks[bank]
        cc = pl.multiple_of(jnp.minimum(c, cmax) * 128, 128)
        ch = pl.multiple_of(
            (jnp.minimum(c, cmax) + CT_HALF) * 128, 128)
        pltpu.async_copy(it3_hbm.at[:, :, pl.ds(cc, 128)], plo, si)
        pltpu.async_copy(it3_hbm.at[:, :, pl.ds(ch, 128)], phi, si)

    def wait_in(bank):
        plo, phi, _, si, _ = banks[bank]
        pltpu.make_async_copy(
            it3_hbm.at[:, :, pl.ds(0, 128)], plo, si).wait()
        pltpu.make_async_copy(
            it3_hbm.at[:, :, pl.ds(0, 128)], phi, si).wait()

    def wait_out(bank):
        _, _, ob, _, so = banks[bank]
        pltpu.make_async_copy(ob, out_hbm.at[pl.ds(0, 128)], so).wait()

    def transpose_panel(pv, ob, half):
        def blk_body(blk, carry):
            fg = blk >> 3
            gi = blk & 7
            xs = [pv[2 * fg + (k >> 3), k & 7, pl.ds(gi * L, L)]
                  for k in range(L)]
            ys = _transpose_block(xs, lane)
            for j in range(L):
                ob[gi * L + j, pl.ds(half * 64 + fg * L, L)] = ys[j]
            return carry
        lax.fori_loop(0, 32, blk_body, 0)

    fire(base, 0)
    fire(base + 1, 1)

    def iter_body(q, carry):
        for bank in range(2):
            plo, phi, ob, _, so = banks[bank]
            c = base + 2 * q + bank
            wait_in(bank)

            @pl.when(q > 0)
            def _():
                wait_out(bank)

            transpose_panel(plo, ob, 0)
            transpose_panel(phi, ob, 1)
            cc = pl.multiple_of(jnp.minimum(c, cmax) * 128, 128)
            pltpu.async_copy(ob, out_hbm.at[pl.ds(cc, 128)], so)
            fire(c + 2, bank)
        return carry

    lax.fori_loop(0, _PAIRS_PW // 2, iter_body, 0)

    for bank in range(2):
        wait_in(bank)
        wait_out(bank)


@functools.partial(
    pl.kernel,
    out_type=jax.ShapeDtypeStruct((NW, BPW), jnp.float32),
    mesh=_mesh,
    scratch_types=[
        pltpu.VMEM((NCHUNK, CHUNK), jnp.int32),
        pltpu.VMEM((NCHUNK, CHUNK), jnp.int32),
        pltpu.VMEM((NCHUNK, CHUNK), jnp.int32),
        pltpu.VMEM((NCHUNK, CHUNK), jnp.int32),
        pltpu.VMEM((NGROUP, L), jnp.int32),
        pltpu.VMEM((NGROUP, L), jnp.int32),
        pltpu.VMEM((CHUNK, 128), jnp.float32),
        pltpu.VMEM((CHUNK, 128), jnp.float32),
        pltpu.VMEM((CHUNK, 128), jnp.float32),
        pltpu.VMEM((CHUNK, 128), jnp.float32),
        pltpu.VMEM((BPW,), jnp.float32),
        pltpu.VMEM((BPW,), jnp.float32),
        pltpu.VMEM((BPW,), jnp.float32),
        pltpu.SemaphoreType.DMA,
        pltpu.SemaphoreType.DMA,
        pltpu.SemaphoreType.DMA,
    ],
)
def _scores_kernel(uid4_hbm, iid4_hbm, urid_hbm, imid_hbm,
                   uidr_hbm, iidr_hbm, uer_hbm, ier_hbm,
                   ub_hbm, ib_hbm,
                   out_hbm, uid_v, iid_v, urid_v, imid_v,
                   uidr_v, iidr_v, ue_a, ue_b, ie_a, ie_b,
                   ub_v, ib_v, out_v, sem_a, sem_b, sem):
    wid = lax.axis_index("s") * NC + lax.axis_index("c")

    pltpu.sync_copy(uid4_hbm.at[wid], uid_v)
    pltpu.sync_copy(iid4_hbm.at[wid], iid_v)
    pltpu.sync_copy(urid_hbm.at[wid], urid_v)
    pltpu.sync_copy(imid_hbm.at[wid], imid_v)
    pltpu.sync_copy(uidr_hbm.at[wid], uidr_v)
    pltpu.sync_copy(iidr_hbm.at[wid], iidr_v)

    bias_copies = []
    for c in range(NCHUNK):
        sl = pl.ds(c * CHUNK, CHUNK)
        bias_copies.append(
            pltpu.async_copy(ub_hbm.at[uid_v.at[c]], ub_v.at[sl], sem))
        bias_copies.append(
            pltpu.async_copy(ib_hbm.at[iid_v.at[c]], ib_v.at[sl], sem))

    bufs = [(ue_a, ie_a, sem_a), (ue_b, ie_b, sem_b)]

    def fire(c):
        ue, ie, s = bufs[c % 2]
        return (pltpu.async_copy(uer_hbm.at[urid_v.at[c]], ue, s),
                pltpu.async_copy(ier_hbm.at[imid_v.at[c]], ie, s))

    lane = lax.iota(jnp.int32, L)
    half_tc = jnp.full((L,), HALF_TC, jnp.int32)
    half_sc = jnp.full((L,), HALF_SC, jnp.int32)
    tail_b = jnp.full((L,), TAIL_BASE, jnp.int32)
    c64 = jnp.full((L,), 64, jnp.int32)
    c0 = jnp.zeros((L,), jnp.int32)
    c1 = jnp.full((L,), 1, jnp.int32)

    def hsum(v):
        for dist in (8, 4, 2, 1):
            v = v + v.at[lane ^ dist].get(mode="promise_in_bounds")
        return v

    gpc = CHUNK // L

    cur = fire(0)
    for c in range(NCHUNK):
        nxt = fire(c + 1) if c + 1 < NCHUNK else None
        for h in cur:
            h.wait()
        if c == 0:
            for cp in bias_copies:
                cp.wait()
        ue_v, ie_v, _ = bufs[c % 2]

        def body(gi, carry, c=c, ue_v=ue_v, ie_v=ie_v):
            g = c * gpc + gi
            row0 = gi * L
            uid16 = uidr_v[g]
            iid16 = iidr_v[g]
            hvu = jnp.where(uid16 >= half_tc, c64, c0)
            hvi = jnp.where((iid16 >= half_sc) & (iid16 < tail_b), c64, c0)
            res = jnp.zeros((L,), jnp.float32)
            for k in range(L):
                hu = hvu[k]
                hi = hvi[k]
                r = row0 + k
                acc = None
                for cc in range(D // L):
                    u = ue_v[r, pl.ds(hu + cc * L, L)]
                    v = ie_v[r, pl.ds(hi + cc * L, L)]
                    term = u * v
                    acc = term if acc is None else acc + term
                res = jnp.where(lane == k, hsum(acc), res)
            sl = pl.ds(g * L, L)
            out_v[sl] = res + ub_v[sl] + ib_v[sl]
            return carry

        lax.fori_loop(0, gpc, body, 0)
        cur = nxt

    pltpu.sync_copy(out_v, out_hbm.at[wid])


def kernel(user_ids, item_ids, user_embed, item_embed, user_bias, item_bias):
    uids = user_ids.astype(jnp.int32)
    iids = item_ids.astype(jnp.int32)

    uer = _relayout_tc(user_embed.T)
    itail = _tail_tc(item_embed.T)
    it3 = item_embed.T.reshape(8, 8, item_embed.shape[0])
    ier = _convert_sc(it3, itail)

    urid = uids % HALF_TC
    is_tail = iids >= TAIL_BASE
    imid = jnp.where(is_tail, HALF_SC + iids - TAIL_BASE, iids % HALF_SC)

    out = _scores_kernel(
        uids.reshape(NW, NCHUNK, CHUNK),
        iids.reshape(NW, NCHUNK, CHUNK),
        urid.reshape(NW, NCHUNK, CHUNK),
        imid.reshape(NW, NCHUNK, CHUNK),
        uids.reshape(NW, NGROUP, L),
        iids.reshape(NW, NGROUP, L),
        uer,
        ier,
        user_bias.reshape(-1),
        item_bias.reshape(-1),
    )
    return out.reshape(B)
